# Initial kernel scaffold; baseline (speedup 1.0000x reference)
import jax
import jax.numpy as jnp
from jax.experimental import pallas as pl


def kernel(x, edge_index, W1_l, b1_l, W1_r, gamma, beta, running_mean, running_var, W2_l, b2_l, W2_r):
    raise NotImplementedError("write your pallas kernel here")


# speedup vs baseline: 7.7202x; 7.7202x over previous
import functools

import jax
import jax.numpy as jnp
from jax import lax
from jax.experimental import pallas as pl
from jax.experimental.pallas import tpu as pltpu
from jax.experimental.pallas import tpu_sc as plsc

N = 10000
E = 320000
D = 128
NC = 2
NS = 16
NW = NC * NS
EPW = E // NW
K = 80
NCHUNK = EPW // K
NPAD = 10240
RPT = NPAD // NS
DPT = NPAD // NS

_MESH = plsc.VectorSubcoreMesh(core_axis_name="c", subcore_axis_name="s")


def _make_sc_pass(with_deg: bool):
    out_type = [jax.ShapeDtypeStruct((NC, NPAD, D), jnp.float32)]
    if with_deg:
        out_type.append(jax.ShapeDtypeStruct((NC, NPAD), jnp.float32))

    scratch = dict(
        src_v=pltpu.VMEM((NCHUNK, K), jnp.int32),
        dst_v=pltpu.VMEM((NCHUNK, K), jnp.int32),
        rows_v=pltpu.VMEM((K, D), jnp.float32),
        ones_v=pltpu.VMEM((K,), jnp.float32),
        acc=pltpu.VMEM_SHARED((NPAD, D), jnp.float32),
        dacc=pltpu.VMEM_SHARED((NPAD,), jnp.float32),
        sem=pltpu.SemaphoreType.DMA,
    )

    @functools.partial(
        pl.kernel, mesh=_MESH, out_type=out_type,
        scratch_types=list(scratch.values()),
    )
    def sc_pass(table_h, src_h, dst_h, z2_h, z1_h, *rest):
        if with_deg:
            part_h, deg_h = rest[0], rest[1]
            rest = rest[2:]
        else:
            part_h = rest[0]
            rest = rest[1:]
        src_v, dst_v, rows_v, ones_v, acc, dacc, sem = rest

        c = lax.axis_index("c")
        s = lax.axis_index("s")
        w = c * NS + s

        pltpu.sync_copy(src_h.at[w], src_v)
        pltpu.sync_copy(dst_h.at[w], dst_v)
        pltpu.sync_copy(z2_h, acc.at[pl.ds(s * RPT, RPT)])
        if with_deg:
            pltpu.sync_copy(z1_h, dacc.at[pl.ds(s * DPT, DPT)])
            for j in range(K // 16):
                ones_v[pl.ds(j * 16, 16)] = jnp.ones((16,), jnp.float32)
        plsc.subcore_barrier()

        def body(g, carry):
            pltpu.async_copy(table_h.at[src_v.at[g]], rows_v, sem).wait()
            pltpu.sync_copy(rows_v, acc.at[dst_v.at[g]], add=True)
            if with_deg:
                pltpu.sync_copy(ones_v, dacc.at[dst_v.at[g]], add=True)
            return carry

        lax.fori_loop(0, NCHUNK, body, 0)
        plsc.subcore_barrier()

        pltpu.sync_copy(acc.at[pl.ds(s * RPT, RPT)],
                        part_h.at[c, pl.ds(s * RPT, RPT)])
        if with_deg:
            pltpu.sync_copy(dacc.at[pl.ds(s * DPT, DPT)],
                            deg_h.at[c, pl.ds(s * DPT, DPT)])

    return sc_pass


_sc_pass_deg = _make_sc_pass(True)
_sc_pass = _make_sc_pass(False)

R = 1000
GRID = N // R


def _combine_body(with_bn, p_ref, d_ref, x_ref, wl_ref, b_ref, wr_ref,
                  g_ref, be_ref, m_ref, v_ref, o_ref):
    p = p_ref[0] + p_ref[1]
    d = d_ref[0] + d_ref[1]
    agg = p * (1.0 / jnp.maximum(d, 1.0))
    h = (jnp.dot(agg, wl_ref[...], preferred_element_type=jnp.float32)
         + jnp.dot(x_ref[...], wr_ref[...], preferred_element_type=jnp.float32)
         + b_ref[...])
    if with_bn:
        scale = g_ref[...] * lax.rsqrt(v_ref[...] + 1e-5)
        h = (h - m_ref[...]) * scale + be_ref[...]
        h = jnp.maximum(h, 0.0)
    o_ref[...] = h


def _make_combine(with_bn: bool):
    full = pl.BlockSpec((1, D), lambda i: (0, 0))
    in_specs = [
        pl.BlockSpec((NC, R, D), lambda i: (0, i, 0)),
        pl.BlockSpec((NC, R, 1), lambda i: (0, i, 0)),
        pl.BlockSpec((R, D), lambda i: (i, 0)),
        pl.BlockSpec((D, D), lambda i: (0, 0)),
        full,
        pl.BlockSpec((D, D), lambda i: (0, 0)),
        full, full, full, full,
    ]
    return pl.pallas_call(
        functools.partial(_combine_body, with_bn),
        grid=(GRID,),
        in_specs=in_specs,
        out_specs=pl.BlockSpec((R, D), lambda i: (i, 0)),
        out_shape=jax.ShapeDtypeStruct((N, D), jnp.float32),
    )


_combine_bn = _make_combine(True)
_combine_plain = _make_combine(False)


@jax.jit
def kernel(x, edge_index, W1_l, b1_l, W1_r, gamma, beta,
           running_mean, running_var, W2_l, b2_l, W2_r):
    src = edge_index[0].astype(jnp.int32).reshape(NW, NCHUNK, K)
    dst = edge_index[1].astype(jnp.int32).reshape(NW, NCHUNK, K)
    z2 = jnp.zeros((RPT, D), jnp.float32)
    z1 = jnp.zeros((DPT,), jnp.float32)

    p1, deg = _sc_pass_deg(x, src, dst, z2, z1)
    degc = deg.reshape(NC, NPAD, 1)
    one128 = lambda a: a.reshape(1, D).astype(jnp.float32)

    h = _combine_bn(p1, degc, x, W1_l.T, one128(b1_l), W1_r.T,
                    one128(gamma), one128(beta),
                    one128(running_mean), one128(running_var))

    (p2,) = _sc_pass(h, src, dst, z2, z1)
    zero = jnp.zeros((1, D), jnp.float32)
    out = _combine_plain(p2, degc, h, W2_l.T, one128(b2_l), W2_r.T,
                         zero, zero, zero, zero)
    return out


# speedup vs baseline: 10.1259x; 1.3116x over previous
import functools

import jax
import jax.numpy as jnp
from jax import lax
from jax.experimental import pallas as pl
from jax.experimental.pallas import tpu as pltpu
from jax.experimental.pallas import tpu_sc as plsc

N = 10000
E = 320000
D = 128
NC = 2
NS = 16
NW = NC * NS
EPW = E // NW
K = 100
NCHUNK = EPW // K
NH = 2
HCHUNK = NCHUNK // NH
NPAD = 10112
RPT = NPAD // NS
DNPAD = 10240
DPT = DNPAD // NS

_MESH = plsc.VectorSubcoreMesh(core_axis_name="c", subcore_axis_name="s")


def _make_sc_pass(with_deg: bool):
    out_type = [jax.ShapeDtypeStruct((NC, NPAD, D), jnp.float32)]
    if with_deg:
        out_type.append(jax.ShapeDtypeStruct((NC * DNPAD,), jnp.float32))

    scratch = dict(
        src_v=pltpu.VMEM((HCHUNK, K), jnp.int32),
        dst_v=pltpu.VMEM((HCHUNK, K), jnp.int32),
        rows0=pltpu.VMEM((K, D), jnp.float32),
        rows1=pltpu.VMEM((K, D), jnp.float32),
        ones_v=pltpu.VMEM((128,), jnp.float32),
        acc=pltpu.VMEM_SHARED((NPAD, D), jnp.float32),
        dacc=pltpu.VMEM_SHARED((DNPAD,), jnp.float32),
        gsem0=pltpu.SemaphoreType.DMA,
        gsem1=pltpu.SemaphoreType.DMA,
        ssem0=pltpu.SemaphoreType.DMA,
        ssem1=pltpu.SemaphoreType.DMA,
    )

    @functools.partial(
        pl.kernel, mesh=_MESH, out_type=out_type,
        scratch_types=list(scratch.values()),
    )
    def sc_pass(table_h, src_h, dst_h, z2_h, z1_h, *rest):
        if with_deg:
            part_h, deg_h = rest[0], rest[1]
            rest = rest[2:]
        else:
            part_h = rest[0]
            rest = rest[1:]
        (src_v, dst_v, rows0, rows1, ones_v, acc, dacc,
         gsem0, gsem1, ssem0, ssem1) = rest

        c = lax.axis_index("c")
        s = lax.axis_index("s")
        w = c * NS + s

        pltpu.sync_copy(z2_h, acc.at[pl.ds(s * RPT, RPT)])
        if with_deg:
            pltpu.sync_copy(z1_h, dacc.at[pl.ds(s * DPT, DPT)])
            for j in range(8):
                ones_v[pl.ds(j * 16, 16)] = jnp.ones((16,), jnp.float32)
        plsc.subcore_barrier()

        ones_k = ones_v.at[pl.ds(0, K)]
        half = HCHUNK // 2

        def gather(g, buf, sem):
            return pltpu.async_copy(table_h.at[src_v.at[g]], buf, sem)

        for hh in range(NH):
            pltpu.sync_copy(src_h.at[w, hh], src_v)
            pltpu.sync_copy(dst_h.at[w, hh], dst_v)
            gather(0, rows0, gsem0)
            gather(1, rows1, gsem1)

            def body(i, carry):
                g = 2 * i
                pltpu.make_async_copy(
                    table_h.at[src_v.at[g]], rows0, gsem0).wait()
                sc0 = pltpu.async_copy(rows0, acc.at[dst_v.at[g]], ssem0,
                                       add=True)
                if with_deg:
                    pltpu.sync_copy(ones_k, dacc.at[dst_v.at[g]], add=True)
                pltpu.make_async_copy(
                    table_h.at[src_v.at[g + 1]], rows1, gsem1).wait()
                sc1 = pltpu.async_copy(rows1, acc.at[dst_v.at[g + 1]], ssem1,
                                       add=True)
                if with_deg:
                    pltpu.sync_copy(ones_k, dacc.at[dst_v.at[g + 1]],
                                    add=True)
                sc0.wait()

                @pl.when(i < half - 1)
                def _():
                    gather(g + 2, rows0, gsem0)
                sc1.wait()

                @pl.when(i < half - 1)
                def _():
                    gather(g + 3, rows1, gsem1)
                return carry

            lax.fori_loop(0, half, body, 0)
        plsc.subcore_barrier()

        pltpu.sync_copy(acc.at[pl.ds(s * RPT, RPT)],
                        part_h.at[c, pl.ds(s * RPT, RPT)])
        if with_deg:
            pltpu.sync_copy(dacc.at[pl.ds(s * DPT, DPT)],
                            deg_h.at[pl.ds(c * DNPAD + s * DPT, DPT)])

    return sc_pass


_sc_pass_deg = _make_sc_pass(True)
_sc_pass = _make_sc_pass(False)

R = 1000
GRID = N // R


def _combine_body(with_bn, p_ref, d_ref, x_ref, wl_ref, b_ref, wr_ref,
                  g_ref, be_ref, m_ref, v_ref, o_ref):
    p = p_ref[0] + p_ref[1]
    d = d_ref[0] + d_ref[1]
    agg = p * (1.0 / jnp.maximum(d, 1.0))
    h = (jnp.dot(agg, wl_ref[...], preferred_element_type=jnp.float32)
         + jnp.dot(x_ref[...], wr_ref[...], preferred_element_type=jnp.float32)
         + b_ref[...])
    if with_bn:
        scale = g_ref[...] * lax.rsqrt(v_ref[...] + 1e-5)
        h = (h - m_ref[...]) * scale + be_ref[...]
        h = jnp.maximum(h, 0.0)
    o_ref[...] = h


def _make_combine(with_bn: bool):
    full = pl.BlockSpec((1, D), lambda i: (0, 0))
    in_specs = [
        pl.BlockSpec((NC, R, D), lambda i: (0, i, 0)),
        pl.BlockSpec((NC, R, 1), lambda i: (0, i, 0)),
        pl.BlockSpec((R, D), lambda i: (i, 0)),
        pl.BlockSpec((D, D), lambda i: (0, 0)),
        full,
        pl.BlockSpec((D, D), lambda i: (0, 0)),
        full, full, full, full,
    ]
    return pl.pallas_call(
        functools.partial(_combine_body, with_bn),
        grid=(GRID,),
        in_specs=in_specs,
        out_specs=pl.BlockSpec((R, D), lambda i: (i, 0)),
        out_shape=jax.ShapeDtypeStruct((N, D), jnp.float32),
    )


_combine_bn = _make_combine(True)
_combine_plain = _make_combine(False)


@jax.jit
def kernel(x, edge_index, W1_l, b1_l, W1_r, gamma, beta,
           running_mean, running_var, W2_l, b2_l, W2_r):
    src = edge_index[0].astype(jnp.int32).reshape(NW, NH, HCHUNK, K)
    dst = edge_index[1].astype(jnp.int32).reshape(NW, NH, HCHUNK, K)
    z2 = jnp.zeros((RPT, D), jnp.float32)
    z1 = jnp.zeros((DPT,), jnp.float32)

    p1, deg = _sc_pass_deg(x, src, dst, z2, z1)
    degc = deg.reshape(NC, DNPAD, 1)
    one128 = lambda a: a.reshape(1, D).astype(jnp.float32)

    h = _combine_bn(p1, degc, x, W1_l.T, one128(b1_l), W1_r.T,
                    one128(gamma), one128(beta),
                    one128(running_mean), one128(running_var))

    (p2,) = _sc_pass(h, src, dst, z2, z1)
    zero = jnp.zeros((1, D), jnp.float32)
    out = _combine_plain(p2, degc, h, W2_l.T, one128(b2_l), W2_r.T,
                         zero, zero, zero, zero)
    return out


# speedup vs baseline: 10.2383x; 1.0111x over previous
import functools

import jax
import jax.numpy as jnp
from jax import lax
from jax.experimental import pallas as pl
from jax.experimental.pallas import tpu as pltpu
from jax.experimental.pallas import tpu_sc as plsc

N = 10000
E = 320000
D = 128
NC = 2
NS = 16
NW = NC * NS
EPW = E // NW
K = 125
NCHUNK = EPW // K
NH = 2
HCHUNK = NCHUNK // NH
NPAD = 10112
RPT = NPAD // NS
DNPAD = 10240
DPT = DNPAD // NS

_MESH = plsc.VectorSubcoreMesh(core_axis_name="c", subcore_axis_name="s")


def _make_sc_pass(with_deg: bool):
    out_type = [jax.ShapeDtypeStruct((NC, NPAD, D), jnp.float32)]
    if with_deg:
        out_type.append(jax.ShapeDtypeStruct((NC * DNPAD,), jnp.float32))

    scratch = dict(
        src_v=pltpu.VMEM((HCHUNK, K), jnp.int32),
        dst_v=pltpu.VMEM((HCHUNK, K), jnp.int32),
        rows0=pltpu.VMEM((K, D), jnp.float32),
        rows1=pltpu.VMEM((K, D), jnp.float32),
        ones_v=pltpu.VMEM((128,), jnp.float32),
        acc=pltpu.VMEM_SHARED((NPAD, D), jnp.float32),
        dacc=pltpu.VMEM_SHARED((DNPAD,), jnp.float32),
        gsem0=pltpu.SemaphoreType.DMA,
        gsem1=pltpu.SemaphoreType.DMA,
        ssem0=pltpu.SemaphoreType.DMA,
        ssem1=pltpu.SemaphoreType.DMA,
    )

    @functools.partial(
        pl.kernel, mesh=_MESH, out_type=out_type,
        scratch_types=list(scratch.values()),
    )
    def sc_pass(table_h, src_h, dst_h, z2_h, z1_h, *rest):
        if with_deg:
            part_h, deg_h = rest[0], rest[1]
            rest = rest[2:]
        else:
            part_h = rest[0]
            rest = rest[1:]
        (src_v, dst_v, rows0, rows1, ones_v, acc, dacc,
         gsem0, gsem1, ssem0, ssem1) = rest

        c = lax.axis_index("c")
        s = lax.axis_index("s")
        w = c * NS + s

        pltpu.sync_copy(z2_h, acc.at[pl.ds(s * RPT, RPT)])
        if with_deg:
            pltpu.sync_copy(z1_h, dacc.at[pl.ds(s * DPT, DPT)])
            for j in range(8):
                ones_v[pl.ds(j * 16, 16)] = jnp.ones((16,), jnp.float32)
        plsc.subcore_barrier()

        ones_k = ones_v.at[pl.ds(0, K)]
        half = HCHUNK // 2

        def gather(g, buf, sem):
            return pltpu.async_copy(table_h.at[src_v.at[g]], buf, sem)

        for hh in range(NH):
            pltpu.sync_copy(src_h.at[w, hh], src_v)
            pltpu.sync_copy(dst_h.at[w, hh], dst_v)
            gather(0, rows0, gsem0)
            gather(1, rows1, gsem1)

            def body(i, carry):
                g = 2 * i
                pltpu.make_async_copy(
                    table_h.at[src_v.at[g]], rows0, gsem0).wait()
                sc0 = pltpu.async_copy(rows0, acc.at[dst_v.at[g]], ssem0,
                                       add=True)
                if with_deg:
                    pltpu.sync_copy(ones_k, dacc.at[dst_v.at[g]], add=True)
                pltpu.make_async_copy(
                    table_h.at[src_v.at[g + 1]], rows1, gsem1).wait()
                sc1 = pltpu.async_copy(rows1, acc.at[dst_v.at[g + 1]], ssem1,
                                       add=True)
                if with_deg:
                    pltpu.sync_copy(ones_k, dacc.at[dst_v.at[g + 1]],
                                    add=True)
                sc0.wait()

                @pl.when(i < half - 1)
                def _():
                    gather(g + 2, rows0, gsem0)
                sc1.wait()

                @pl.when(i < half - 1)
                def _():
                    gather(g + 3, rows1, gsem1)
                return carry

            lax.fori_loop(0, half, body, 0)
        plsc.subcore_barrier()

        pltpu.sync_copy(acc.at[pl.ds(s * RPT, RPT)],
                        part_h.at[c, pl.ds(s * RPT, RPT)])
        if with_deg:
            pltpu.sync_copy(dacc.at[pl.ds(s * DPT, DPT)],
                            deg_h.at[pl.ds(c * DNPAD + s * DPT, DPT)])

    return sc_pass


_sc_pass_deg = _make_sc_pass(True)
_sc_pass = _make_sc_pass(False)

R = 1000
GRID = N // R


def _combine_body(with_bn, p_ref, d_ref, x_ref, wl_ref, b_ref, wr_ref,
                  g_ref, be_ref, m_ref, v_ref, o_ref):
    p = p_ref[0] + p_ref[1]
    d = d_ref[0] + d_ref[1]
    agg = p * (1.0 / jnp.maximum(d, 1.0))
    h = (jnp.dot(agg, wl_ref[...], preferred_element_type=jnp.float32)
         + jnp.dot(x_ref[...], wr_ref[...], preferred_element_type=jnp.float32)
         + b_ref[...])
    if with_bn:
        scale = g_ref[...] * lax.rsqrt(v_ref[...] + 1e-5)
        h = (h - m_ref[...]) * scale + be_ref[...]
        h = jnp.maximum(h, 0.0)
    o_ref[...] = h


def _make_combine(with_bn: bool):
    full = pl.BlockSpec((1, D), lambda i: (0, 0))
    in_specs = [
        pl.BlockSpec((NC, R, D), lambda i: (0, i, 0)),
        pl.BlockSpec((NC, R, 1), lambda i: (0, i, 0)),
        pl.BlockSpec((R, D), lambda i: (i, 0)),
        pl.BlockSpec((D, D), lambda i: (0, 0)),
        full,
        pl.BlockSpec((D, D), lambda i: (0, 0)),
        full, full, full, full,
    ]
    return pl.pallas_call(
        functools.partial(_combine_body, with_bn),
        grid=(GRID,),
        in_specs=in_specs,
        out_specs=pl.BlockSpec((R, D), lambda i: (i, 0)),
        out_shape=jax.ShapeDtypeStruct((N, D), jnp.float32),
    )


_combine_bn = _make_combine(True)
_combine_plain = _make_combine(False)


@jax.jit
def kernel(x, edge_index, W1_l, b1_l, W1_r, gamma, beta,
           running_mean, running_var, W2_l, b2_l, W2_r):
    src = edge_index[0].astype(jnp.int32).reshape(NW, NH, HCHUNK, K)
    dst = edge_index[1].astype(jnp.int32).reshape(NW, NH, HCHUNK, K)
    z2 = jnp.zeros((RPT, D), jnp.float32)
    z1 = jnp.zeros((DPT,), jnp.float32)

    p1, deg = _sc_pass_deg(x, src, dst, z2, z1)
    degc = deg.reshape(NC, DNPAD, 1)
    one128 = lambda a: a.reshape(1, D).astype(jnp.float32)

    h = _combine_bn(p1, degc, x, W1_l.T, one128(b1_l), W1_r.T,
                    one128(gamma), one128(beta),
                    one128(running_mean), one128(running_var))

    (p2,) = _sc_pass(h, src, dst, z2, z1)
    zero = jnp.zeros((1, D), jnp.float32)
    out = _combine_plain(p2, degc, h, W2_l.T, one128(b2_l), W2_r.T,
                         zero, zero, zero, zero)
    return out
